# baseline (speedup 1.0000x reference)
import functools

import jax
import jax.numpy as jnp
from jax import lax
from jax.experimental import pallas as pl
from jax.experimental.pallas import tpu as pltpu
from jax.experimental.pallas import tpu_sc as plsc

NC = 2
NS = 16
NW = NC * NS
L = 16
CHUNK = 64


@functools.partial(jax.jit, static_argnames=("epw", "d"))
def _decode(cidx, z_cat, *, epw, d):
    e_pad = cidx.shape[0] // 2
    n_chunks = epw // CHUNK
    mesh = plsc.VectorSubcoreMesh(
        core_axis_name="c", subcore_axis_name="s", num_cores=NC,
        num_subcores=NS)

    @functools.partial(
        pl.kernel,
        out_type=jax.ShapeDtypeStruct((e_pad,), jnp.float32),
        mesh=mesh,
        compiler_params=pltpu.CompilerParams(needs_layout_passes=False),
        scratch_types=[
            pltpu.VMEM((2 * epw,), jnp.int32),
            pltpu.VMEM((2 * CHUNK, d), jnp.float32),
            pltpu.VMEM((2 * CHUNK, d), jnp.float32),
            pltpu.VMEM((epw + L * L,), jnp.float32),
            pltpu.SemaphoreType.DMA,
            pltpu.SemaphoreType.DMA,
        ],
    )
    def sc_decode(cidx_hbm, zcat_hbm, out_hbm, cidx_v, rows0, rows1, outv,
                  sem0, sem1):
        rows = (rows0, rows1)
        sems = (sem0, sem1)
        wid = lax.axis_index("s") * NC + lax.axis_index("c")
        base = wid * epw
        pltpu.sync_copy(cidx_hbm.at[pl.ds(2 * base, 2 * epw)], cidx_v)
        tb = lax.iota(jnp.int32, L) * L + epw

        def start_gather(ci, b):
            cb = 2 * ci * CHUNK
            pltpu.async_copy(
                zcat_hbm.at[cidx_v.at[pl.ds(cb, 2 * CHUNK)]], rows[b],
                sems[b])

        def wait_gather(ci, b):
            cb = 2 * ci * CHUNK
            pltpu.make_async_copy(
                zcat_hbm.at[cidx_v.at[pl.ds(cb, 2 * CHUNK)]], rows[b],
                sems[b]).wait()

        def chunk_body(ci, carry):
            b = 0
            start_gather(ci, b)
            wait_gather(ci, b)

            def group_body(g, carry2):
                gb = ci * CHUNK + g * L
                for e16 in range(L):
                    e = 2 * (g * L + e16)
                    acc = (rows[b][e, pl.ds(0, L)]
                           * rows[b][e + 1, pl.ds(0, L)])
                    for j in range(1, d // L):
                        acc = acc + (rows[b][e, pl.ds(j * L, L)]
                                     * rows[b][e + 1, pl.ds(j * L, L)])
                    plsc.store_scatter(outv, [tb + e16], acc)
                dots = outv[pl.ds(epw, L)]
                for l in range(1, L):
                    dots = dots + outv[pl.ds(epw + l * L, L)]
                outv[pl.ds(gb, L)] = dots
                return carry2

            lax.fori_loop(0, CHUNK // L, group_body, 0)
            return carry

        lax.fori_loop(0, n_chunks, chunk_body, 0)
        pltpu.sync_copy(outv.at[pl.ds(0, epw)], out_hbm.at[pl.ds(base, epw)])

    return sc_decode(cidx, z_cat)


def kernel(z_src, z_dst, edge_label_index):
    src = edge_label_index[0].astype(jnp.int32)
    dst = edge_label_index[1].astype(jnp.int32)
    e = src.shape[0]
    v, d = z_src.shape
    grain = NW * CHUNK
    e_pad = -(-e // grain) * grain
    if e_pad != e:
        src = jnp.concatenate([src, jnp.zeros((e_pad - e,), jnp.int32)])
        dst = jnp.concatenate([dst, jnp.zeros((e_pad - e,), jnp.int32)])
    cidx = jnp.stack([src, dst + v], axis=1).reshape(2 * e_pad)
    z_cat = jnp.concatenate([z_src, z_dst], axis=0)
    out = _decode(cidx, z_cat, epw=e_pad // NW, d=d)
    return out[:e]


# speedup vs baseline: 1.8839x; 1.8839x over previous
import functools

import jax
import jax.numpy as jnp
from jax import lax
from jax.experimental import pallas as pl
from jax.experimental.pallas import tpu as pltpu
from jax.experimental.pallas import tpu_sc as plsc

NC = 2
NS = 16
NW = NC * NS
L = 16
CHUNK = 64


@functools.partial(jax.jit, static_argnames=("epw", "d"))
def _decode(src_idx, dst_idx, z_src, z_dst, *, epw, d):
    e_pad = src_idx.shape[0]
    n_chunks = epw // CHUNK
    mesh = plsc.VectorSubcoreMesh(
        core_axis_name="c", subcore_axis_name="s", num_cores=NC,
        num_subcores=NS)

    @functools.partial(
        pl.kernel,
        out_type=jax.ShapeDtypeStruct((e_pad,), jnp.float32),
        mesh=mesh,
        compiler_params=pltpu.CompilerParams(needs_layout_passes=False),
        scratch_types=[
            pltpu.VMEM((epw,), jnp.int32),
            pltpu.VMEM((epw,), jnp.int32),
            pltpu.VMEM((2 * CHUNK, d), jnp.float32),
            pltpu.VMEM((2 * CHUNK, d), jnp.float32),
            pltpu.VMEM((epw + L * L,), jnp.float32),
            pltpu.SemaphoreType.DMA,
            pltpu.SemaphoreType.DMA,
        ],
    )
    def sc_decode(src_hbm, dst_hbm, zsrc_hbm, zdst_hbm, out_hbm,
                  sidx_v, didx_v, srows, drows, outv, sem_s, sem_d):
        wid = lax.axis_index("s") * NC + lax.axis_index("c")
        base = wid * epw
        pltpu.sync_copy(src_hbm.at[pl.ds(base, epw)], sidx_v)
        pltpu.sync_copy(dst_hbm.at[pl.ds(base, epw)], didx_v)
        tb = lax.iota(jnp.int32, L) * L + epw

        def gathers(ci):
            cb = ci * CHUNK
            half = (ci % 2) * CHUNK
            cp_s = pltpu.make_async_copy(
                zsrc_hbm.at[sidx_v.at[pl.ds(cb, CHUNK)]],
                srows.at[pl.ds(half, CHUNK), :], sem_s)
            cp_d = pltpu.make_async_copy(
                zdst_hbm.at[didx_v.at[pl.ds(cb, CHUNK)]],
                drows.at[pl.ds(half, CHUNK), :], sem_d)
            return cp_s, cp_d

        def prime(ci):
            cp_s, cp_d = gathers(ci)
            cp_s.start()
            cp_d.start()

        prime(0)

        def chunk_body(ci, carry):
            @pl.when(ci + 1 < n_chunks)
            def _():
                prime(ci + 1)

            cp_s, cp_d = gathers(ci)
            cp_s.wait()
            cp_d.wait()
            half = (ci % 2) * CHUNK

            def group_body(g, carry2):
                gb = ci * CHUNK + g * L
                for e16 in range(L):
                    e = half + g * L + e16
                    acc = srows[e, pl.ds(0, L)] * drows[e, pl.ds(0, L)]
                    for j in range(1, d // L):
                        acc = acc + (srows[e, pl.ds(j * L, L)]
                                     * drows[e, pl.ds(j * L, L)])
                    plsc.store_scatter(outv, [tb + e16], acc)
                dots = outv[pl.ds(epw, L)]
                for l in range(1, L):
                    dots = dots + outv[pl.ds(epw + l * L, L)]
                outv[pl.ds(gb, L)] = dots
                return carry2

            lax.fori_loop(0, CHUNK // L, group_body, 0)
            return carry

        lax.fori_loop(0, n_chunks, chunk_body, 0)
        pltpu.sync_copy(outv.at[pl.ds(0, epw)], out_hbm.at[pl.ds(base, epw)])

    return sc_decode(src_idx, dst_idx, z_src, z_dst)


def kernel(z_src, z_dst, edge_label_index):
    src = edge_label_index[0].astype(jnp.int32)
    dst = edge_label_index[1].astype(jnp.int32)
    e = src.shape[0]
    d = z_src.shape[1]
    grain = NW * CHUNK
    e_pad = -(-e // grain) * grain
    if e_pad != e:
        src = jnp.concatenate([src, jnp.zeros((e_pad - e,), jnp.int32)])
        dst = jnp.concatenate([dst, jnp.zeros((e_pad - e,), jnp.int32)])
    out = _decode(src, dst, z_src, z_dst, epw=e_pad // NW, d=d)
    return out[:e]
